# baseline (speedup 1.0000x reference)
import functools

import numpy as np
import jax
import jax.numpy as jnp
from jax import lax
from jax.experimental import pallas as pl
from jax.experimental.pallas import tpu as pltpu
from jax.experimental.pallas import tpu_sc as plsc

_NB = 8192
_CD = 256
_T = 2048
_N = 8
_SCALES = (1, 2, 4, 8)
_CHUNK = 512


def _make_dist_call(scale):
    t_in = _T // scale

    def body(res_ref, cb_ref, idx_ref):
        res = res_ref[0]
        if scale == 1:
            down = res
        else:
            down = res.reshape(t_in, scale, _CD).mean(axis=1)
        rowsum = jnp.sum(down * down, axis=1, keepdims=True)
        ones = jnp.ones((1, _CD), jnp.float32)
        best_d = jnp.full((t_in, 1), jnp.inf, jnp.float32)
        best_i = jnp.zeros((t_in, 1), jnp.int32)
        for c0 in range(0, _NB, _CHUNK):
            cb = cb_ref[c0:c0 + _CHUNK, :]
            cross = lax.dot_general(down, cb, (((1,), (1,)), ((), ())),
                                    preferred_element_type=jnp.float32)
            csum = lax.dot_general(ones, cb * cb, (((1,), (1,)), ((), ())),
                                   preferred_element_type=jnp.float32,
                                   precision=lax.Precision.HIGHEST)
            dist = rowsum - 2.0 * cross + csum
            lmin = jnp.min(dist, axis=1, keepdims=True)
            ii = lax.broadcasted_iota(jnp.int32, (t_in, _CHUNK), 1)
            li = jnp.min(jnp.where(dist == lmin, ii, _NB), axis=1,
                         keepdims=True) + c0
            take = lmin < best_d
            best_d = jnp.where(take, lmin, best_d)
            best_i = jnp.where(take, li, best_i)
        idx_ref[0] = best_i

    return pl.pallas_call(
        body,
        grid=(_N,),
        in_specs=[pl.BlockSpec((1, _T, _CD), lambda n: (n, 0, 0)),
                  pl.BlockSpec((_NB, _CD), lambda n: (0, 0))],
        out_specs=pl.BlockSpec((1, t_in, 1), lambda n: (n, 0, 0)),
        out_shape=jax.ShapeDtypeStruct((_N, t_in, 1), jnp.int32),
    )


def _make_gather_call(n_rows):
    info = plsc.get_sparse_core_info()
    nw = info.num_cores * info.num_subcores
    b_per_w = n_rows // nw
    ch = min(b_per_w, 128)
    nch = b_per_w // ch
    mesh = plsc.VectorSubcoreMesh(core_axis_name="c", subcore_axis_name="s")

    @functools.partial(
        pl.kernel,
        mesh=mesh,
        out_type=jax.ShapeDtypeStruct((n_rows, _CD), jnp.float32),
        scratch_types=[pltpu.VMEM((ch,), jnp.int32),
                       pltpu.VMEM((ch, _CD), jnp.float32),
                       pltpu.SemaphoreType.DMA],
    )
    def k(idx_hbm, table_hbm, out_hbm, idx_v, rows_v, sem):
        wid = lax.axis_index("s") * info.num_cores + lax.axis_index("c")
        base = wid * b_per_w
        for i in range(nch):
            off = base + i * ch
            pltpu.sync_copy(idx_hbm.at[pl.ds(off, ch)], idx_v)
            pltpu.async_copy(table_hbm.at[idx_v], rows_v, sem).wait()
            pltpu.sync_copy(rows_v, out_hbm.at[pl.ds(off, ch)])

    return k


def _gather_rows(idx_flat, codebook):
    return _make_gather_call(idx_flat.shape[0])(idx_flat, codebook)


def _upsample_weights(t_in, t_out):
    scale = t_in / t_out
    coords = (np.arange(t_out, dtype=np.float64) + 0.5) * scale - 0.5
    coords = np.clip(coords, 0.0, t_in - 1.0)
    lo = np.floor(coords).astype(np.int64)
    hi = np.minimum(lo + 1, t_in - 1)
    w = (coords - lo).astype(np.float32)
    wm = np.zeros((t_out, t_in), np.float32)
    wm[np.arange(t_out), lo] += (1.0 - w)
    wm[np.arange(t_out), hi] += w
    return jnp.asarray(wm)


def _make_update1_call():
    def body(res_ref, xd_ref, out_ref):
        out_ref[0] = res_ref[0] - xd_ref[0]

    spec = pl.BlockSpec((1, _T, _CD), lambda n: (n, 0, 0))
    return pl.pallas_call(
        body, grid=(_N,),
        in_specs=[spec, spec], out_specs=spec,
        out_shape=jax.ShapeDtypeStruct((_N, _T, _CD), jnp.float32),
    )


def _make_update_call(scale, last):
    t_in = _T // scale

    def body(*refs):
        if last:
            res_ref, xd_ref, w_ref, x_ref, out_ref = refs
        else:
            res_ref, xd_ref, w_ref, out_ref = refs
        up = lax.dot_general(w_ref[...], xd_ref[0], (((1,), (0,)), ((), ())),
                             preferred_element_type=jnp.float32,
                             precision=lax.Precision.HIGHEST)
        if last:
            out_ref[0] = x_ref[0] - (res_ref[0] - up)
        else:
            out_ref[0] = res_ref[0] - up

    spec_t = pl.BlockSpec((1, _T, _CD), lambda n: (n, 0, 0))
    in_specs = [spec_t,
                pl.BlockSpec((1, t_in, _CD), lambda n: (n, 0, 0)),
                pl.BlockSpec((_T, t_in), lambda n: (0, 0))]
    if last:
        in_specs.append(spec_t)
    return pl.pallas_call(
        body, grid=(_N,),
        in_specs=in_specs, out_specs=spec_t,
        out_shape=jax.ShapeDtypeStruct((_N, _T, _CD), jnp.float32),
    )


def kernel(x, codebook, return_latent):
    del return_latent
    x_t = jnp.transpose(x, (0, 2, 1))
    res = x_t
    idx_outs = []
    fhat_t = None
    for s in _SCALES:
        t_in = _T // s
        idx3 = _make_dist_call(s)(res, codebook)
        idx_outs.append(idx3.reshape(_N, t_in))
        xd = _gather_rows(idx3.reshape(_N * t_in), codebook).reshape(_N, t_in, _CD)
        if s == 1:
            res = _make_update1_call()(res, xd)
        elif s != _SCALES[-1]:
            res = _make_update_call(s, False)(res, xd, _upsample_weights(t_in, _T))
        else:
            fhat_t = _make_update_call(s, True)(res, xd, _upsample_weights(t_in, _T), x_t)
    f_hat = jnp.transpose(fhat_t, (0, 2, 1))
    return (*idx_outs, f_hat)


# speedup vs baseline: 1.0658x; 1.0658x over previous
import functools

import numpy as np
import jax
import jax.numpy as jnp
from jax import lax
from jax.experimental import pallas as pl
from jax.experimental.pallas import tpu as pltpu
from jax.experimental.pallas import tpu_sc as plsc

_NB = 8192
_CD = 256
_T = 2048
_N = 8
_SCALES = (1, 2, 4, 8)
_CHUNK = 512


def _make_dist_call(scale):
    t_in = _T // scale

    def body(res_ref, cb_ref, idx_ref):
        res = res_ref[0]
        if scale == 1:
            down = res
        else:
            down = res.reshape(t_in, scale, _CD).mean(axis=1)
        rowsum = jnp.sum(down * down, axis=1, keepdims=True)
        ones = jnp.ones((1, _CD), jnp.float32)
        best_d = jnp.full((t_in, 1), jnp.inf, jnp.float32)
        best_i = jnp.zeros((t_in, 1), jnp.int32)
        for c0 in range(0, _NB, _CHUNK):
            cb = cb_ref[c0:c0 + _CHUNK, :]
            cross = lax.dot_general(down, cb, (((1,), (1,)), ((), ())),
                                    preferred_element_type=jnp.float32)
            csum = lax.dot_general(ones, cb * cb, (((1,), (1,)), ((), ())),
                                   preferred_element_type=jnp.float32,
                                   precision=lax.Precision.HIGHEST)
            dist = rowsum - 2.0 * cross + csum
            lmin = jnp.min(dist, axis=1, keepdims=True)
            ii = lax.broadcasted_iota(jnp.int32, (t_in, _CHUNK), 1)
            li = jnp.min(jnp.where(dist == lmin, ii, _NB), axis=1,
                         keepdims=True) + c0
            take = lmin < best_d
            best_d = jnp.where(take, lmin, best_d)
            best_i = jnp.where(take, li, best_i)
        idx_ref[0] = best_i

    return pl.pallas_call(
        body,
        grid=(_N,),
        in_specs=[pl.BlockSpec((1, _T, _CD), lambda n: (n, 0, 0)),
                  pl.BlockSpec((_NB, _CD), lambda n: (0, 0))],
        out_specs=pl.BlockSpec((1, t_in, 1), lambda n: (n, 0, 0)),
        out_shape=jax.ShapeDtypeStruct((_N, t_in, 1), jnp.int32),
    )


def _make_gather_call(n_rows):
    info = plsc.get_sparse_core_info()
    nw = info.num_cores * info.num_subcores
    b_per_w = n_rows // nw
    ch = min(b_per_w, 128)
    nch = b_per_w // ch
    mesh = plsc.VectorSubcoreMesh(core_axis_name="c", subcore_axis_name="s")

    @functools.partial(
        pl.kernel,
        mesh=mesh,
        out_type=jax.ShapeDtypeStruct((n_rows, _CD), jnp.float32),
        scratch_types=[pltpu.VMEM((ch,), jnp.int32),
                       pltpu.VMEM((ch, _CD), jnp.float32),
                       pltpu.SemaphoreType.DMA],
    )
    def k(idx_hbm, table_hbm, out_hbm, idx_v, rows_v, sem):
        wid = lax.axis_index("s") * info.num_cores + lax.axis_index("c")
        base = wid * b_per_w
        for i in range(nch):
            off = base + i * ch
            pltpu.sync_copy(idx_hbm.at[pl.ds(off, ch)], idx_v)
            pltpu.async_copy(table_hbm.at[idx_v], rows_v, sem).wait()
            pltpu.sync_copy(rows_v, out_hbm.at[pl.ds(off, ch)])

    return k


def _gather_rows(idx_flat, codebook):
    return _make_gather_call(idx_flat.shape[0])(idx_flat, codebook)


def _upsample_weights(t_in, t_out):
    scale = t_in / t_out
    coords = (np.arange(t_out, dtype=np.float64) + 0.5) * scale - 0.5
    coords = np.clip(coords, 0.0, t_in - 1.0)
    lo = np.floor(coords).astype(np.int64)
    hi = np.minimum(lo + 1, t_in - 1)
    w = (coords - lo).astype(np.float32)
    wm = np.zeros((t_out, t_in), np.float32)
    wm[np.arange(t_out), lo] += (1.0 - w)
    wm[np.arange(t_out), hi] += w
    return jnp.asarray(wm)


def _make_update1_call():
    def body(res_ref, xd_ref, out_ref):
        out_ref[0] = res_ref[0] - xd_ref[0]

    spec = pl.BlockSpec((1, _T, _CD), lambda n: (n, 0, 0))
    return pl.pallas_call(
        body, grid=(_N,),
        in_specs=[spec, spec], out_specs=spec,
        out_shape=jax.ShapeDtypeStruct((_N, _T, _CD), jnp.float32),
    )


def _make_update_call(scale, last):
    t_in = _T // scale

    def body(*refs):
        if last:
            res_ref, xd_ref, w_ref, x_ref, out_ref = refs
        else:
            res_ref, xd_ref, w_ref, out_ref = refs
        up = lax.dot_general(w_ref[...], xd_ref[0], (((1,), (0,)), ((), ())),
                             preferred_element_type=jnp.float32)
        if last:
            out_ref[0] = x_ref[0] - (res_ref[0] - up)
        else:
            out_ref[0] = res_ref[0] - up

    spec_t = pl.BlockSpec((1, _T, _CD), lambda n: (n, 0, 0))
    in_specs = [spec_t,
                pl.BlockSpec((1, t_in, _CD), lambda n: (n, 0, 0)),
                pl.BlockSpec((_T, t_in), lambda n: (0, 0))]
    if last:
        in_specs.append(spec_t)
    return pl.pallas_call(
        body, grid=(_N,),
        in_specs=in_specs, out_specs=spec_t,
        out_shape=jax.ShapeDtypeStruct((_N, _T, _CD), jnp.float32),
    )


def kernel(x, codebook, return_latent):
    del return_latent
    x_t = jnp.transpose(x, (0, 2, 1))
    res = x_t
    idx_outs = []
    fhat_t = None
    for s in _SCALES:
        t_in = _T // s
        idx3 = _make_dist_call(s)(res, codebook)
        idx_outs.append(idx3.reshape(_N, t_in))
        xd = _gather_rows(idx3.reshape(_N * t_in), codebook).reshape(_N, t_in, _CD)
        if s == 1:
            res = _make_update1_call()(res, xd)
        elif s != _SCALES[-1]:
            res = _make_update_call(s, False)(res, xd, _upsample_weights(t_in, _T))
        else:
            fhat_t = _make_update_call(s, True)(res, xd, _upsample_weights(t_in, _T), x_t)
    f_hat = jnp.transpose(fhat_t, (0, 2, 1))
    return (*idx_outs, f_hat)
